# baseline (speedup 1.0000x reference)
import jax
from jax.experimental import pallas as pl
from jax.experimental.pallas import tpu as pltpu

_BLOCK_ROWS = 3336


def _copy_kernel(x_ref, o_ref):
    o_ref[...] = x_ref[...]


def kernel(x, edge_index):
    del edge_index
    n, d = x.shape
    grid = (pl.cdiv(n, _BLOCK_ROWS),)
    return pl.pallas_call(
        _copy_kernel,
        out_shape=jax.ShapeDtypeStruct(x.shape, x.dtype),
        grid=grid,
        in_specs=[pl.BlockSpec((_BLOCK_ROWS, d), lambda i: (i, 0))],
        out_specs=pl.BlockSpec((_BLOCK_ROWS, d), lambda i: (i, 0)),
        compiler_params=pltpu.CompilerParams(
            dimension_semantics=("arbitrary",),
        ),
    )(x)


# speedup vs baseline: 1.1143x; 1.1143x over previous
import jax
from jax.experimental import pallas as pl
from jax.experimental.pallas import tpu as pltpu

_BLOCK_ROWS = 10000


def _copy_kernel(x_ref, o_ref):
    o_ref[...] = x_ref[...]


def kernel(x, edge_index):
    del edge_index
    n, d = x.shape
    grid = (pl.cdiv(n, _BLOCK_ROWS),)
    return pl.pallas_call(
        _copy_kernel,
        out_shape=jax.ShapeDtypeStruct(x.shape, x.dtype),
        grid=grid,
        in_specs=[pl.BlockSpec((_BLOCK_ROWS, d), lambda i: (i, 0))],
        out_specs=pl.BlockSpec((_BLOCK_ROWS, d), lambda i: (i, 0)),
        compiler_params=pltpu.CompilerParams(
            dimension_semantics=("arbitrary",),
        ),
    )(x)
